# baseline (device time: 111583 ns/iter reference)
import jax
import jax.numpy as jnp
from jax import lax
from jax.experimental import pallas as pl
from jax.experimental.pallas import tpu as pltpu

N_DEV = 8
SQ = 2048
D_MODEL = 1024
HQ_PER = 8
DH = 128
D_HEADS = HQ_PER * DH
BLK = SQ // N_DEV
WIN = 128
KW = 512
SCALE = 0.08838834764831843


def kernel(x, Wq, K_ext, V_ext, Wo):
    cdt = jnp.bfloat16

    kc = K_ext.reshape(SQ, HQ_PER * DH)
    vc = V_ext.reshape(SQ, HQ_PER * DH)

    def body(x_hbm, wq_hbm, k_hbm, v_hbm, wo_hbm, out_ref,
             xv, wqv, wov, wqb, wob, kb, vb, kvstage,
             ctx_ref, send_ref, rs_ref, red_ref, ag_ref, acc_ref,
             load_sems, stage_sems, rs_send_sems, rs_recv_sems,
             ag_send_sems, ag_recv_sems):
        me_i = lax.axis_index("i")

        loads = [
            pltpu.make_async_copy(x_hbm.at[0], xv, load_sems.at[0]),
            pltpu.make_async_copy(
                wq_hbm.at[:, pl.ds(me_i * D_HEADS, D_HEADS)], wqv,
                load_sems.at[1]),
            pltpu.make_async_copy(
                wo_hbm.at[pl.ds(me_i * D_HEADS, D_HEADS), :], wov,
                load_sems.at[2]),
        ]
        for ld in loads:
            ld.start()

        def kv_dma(i, slot):
            src = k_hbm if i % 2 == 0 else v_hbm
            h = i // 2
            return pltpu.make_async_copy(
                src.at[:, pl.ds(h * DH, DH)], kvstage.at[slot],
                stage_sems.at[slot])

        kv_dmas = [kv_dma(i, i % 4) for i in range(2 * HQ_PER)]
        for i in range(4):
            kv_dmas[i].start()

        bar = pltpu.get_barrier_semaphore()
        for j in range(1, N_DEV):
            pl.semaphore_signal(
                bar, inc=1,
                device_id=(lax.rem(me_i + j, N_DEV),),
                device_id_type=pl.DeviceIdType.MESH,
            )
        pl.semaphore_wait(bar, N_DEV - 1)

        loads[0].wait()
        loads[1].wait()
        wqb[...] = (wqv[...] * SCALE).astype(cdt)

        rs_rdmas = []
        for j in range(N_DEV):
            b = lax.rem(me_i + j, N_DEV)
            q0 = b * BLK
            kw = jnp.clip(q0 - WIN, 0, SQ - KW)
            kw = pl.multiple_of(kw, 128)
            xb = xv[pl.ds(q0, BLK), :].astype(cdt)
            qb = lax.dot_general(
                xb, wqb[...], (((1,), (0,)), ((), ())),
                preferred_element_type=jnp.float32)
            ri = lax.broadcasted_iota(jnp.int32, (BLK, KW), 0) + q0
            ci = lax.broadcasted_iota(jnp.int32, (BLK, KW), 1) + kw
            mask = jnp.abs(ri - ci) <= WIN
            for h in range(HQ_PER):
                if j == 0:
                    for i in (2 * h, 2 * h + 1):
                        kv_dmas[i].wait()
                        dst = kb if i % 2 == 0 else vb
                        dst[i // 2] = kvstage[i % 4].astype(cdt)
                        if i + 4 < 2 * HQ_PER:
                            kv_dmas[i + 4].start()
                qh = qb[:, h * DH:(h + 1) * DH].astype(cdt)
                ks = kb[h, pl.ds(kw, KW), :]
                s = lax.dot_general(
                    qh, ks, (((1,), (1,)), ((), ())),
                    preferred_element_type=jnp.float32)
                w = jnp.exp(jnp.where(mask, s, -1e9))
                recip = 1.0 / jnp.sum(w, axis=1, keepdims=True)
                vs = vb[h, pl.ds(kw, KW), :]
                ctxh = lax.dot_general(
                    w.astype(cdt), vs, (((1,), (0,)), ((), ())),
                    preferred_element_type=jnp.float32)
                ctx_ref[:, h * DH:(h + 1) * DH] = (ctxh * recip).astype(cdt)
            if j == 0:
                loads[2].wait()
                wob[...] = wov[...].astype(cdt)
            partial = lax.dot_general(
                ctx_ref[...], wob[...], (((1,), (0,)), ((), ())),
                preferred_element_type=jnp.float32)
            out_ref[0, pl.ds(q0, BLK), :] = partial
            if j == 0:
                continue
            send_ref[j - 1, :, :] = partial.astype(cdt)
            rdma = pltpu.make_async_remote_copy(
                src_ref=send_ref.at[j - 1],
                dst_ref=rs_ref.at[j - 1],
                send_sem=rs_send_sems.at[j - 1],
                recv_sem=rs_recv_sems.at[j - 1],
                device_id=(b,),
                device_id_type=pl.DeviceIdType.MESH,
            )
            rdma.start()
            rs_rdmas.append(rdma)

        acc_ref[...] = out_ref[0, pl.ds(me_i * BLK, BLK), :]
        for j in range(1, N_DEV):
            rs_rdmas[j - 1].wait_recv()
            acc_ref[...] += rs_ref[j - 1].astype(jnp.float32)
        out_ref[0, pl.ds(me_i * BLK, BLK), :] = acc_ref[...]
        red_ref[...] = acc_ref[...].astype(cdt)

        ag_rdmas = []
        for j in range(1, N_DEV):
            peer = lax.rem(me_i + j, N_DEV)
            rdma = pltpu.make_async_remote_copy(
                src_ref=red_ref,
                dst_ref=ag_ref.at[j - 1],
                send_sem=ag_send_sems.at[j - 1],
                recv_sem=ag_recv_sems.at[j - 1],
                device_id=(peer,),
                device_id_type=pl.DeviceIdType.MESH,
            )
            rdma.start()
            ag_rdmas.append(rdma)

        for j in range(1, N_DEV):
            ag_rdmas[j - 1].wait_recv()
            src_chunk = lax.rem(me_i + (N_DEV - j), N_DEV)
            out_ref[0, pl.ds(src_chunk * BLK, BLK), :] = ag_ref[j - 1].astype(jnp.float32)

        for r in rs_rdmas + ag_rdmas:
            r.wait_send()

    out = pl.pallas_call(
        body,
        out_shape=jax.ShapeDtypeStruct((1, SQ, D_MODEL), jnp.float32),
        in_specs=[pl.BlockSpec(memory_space=pltpu.MemorySpace.HBM)] * 5,
        out_specs=pl.BlockSpec(memory_space=pltpu.VMEM),
        scratch_shapes=[
            pltpu.VMEM((SQ, D_MODEL), jnp.float32),
            pltpu.VMEM((D_MODEL, D_HEADS), jnp.float32),
            pltpu.VMEM((D_HEADS, D_MODEL), jnp.float32),
            pltpu.VMEM((D_MODEL, D_HEADS), cdt),
            pltpu.VMEM((D_HEADS, D_MODEL), cdt),
            pltpu.VMEM((HQ_PER, SQ, DH), cdt),
            pltpu.VMEM((HQ_PER, SQ, DH), cdt),
            pltpu.VMEM((4, SQ, DH), jnp.float32),
            pltpu.VMEM((BLK, D_HEADS), cdt),
            pltpu.VMEM((N_DEV - 1, BLK, D_MODEL), cdt),
            pltpu.VMEM((N_DEV - 1, BLK, D_MODEL), cdt),
            pltpu.VMEM((BLK, D_MODEL), cdt),
            pltpu.VMEM((N_DEV - 1, BLK, D_MODEL), cdt),
            pltpu.VMEM((BLK, D_MODEL), jnp.float32),
            pltpu.SemaphoreType.DMA((3,)),
            pltpu.SemaphoreType.DMA((4,)),
            pltpu.SemaphoreType.DMA((N_DEV - 1,)),
            pltpu.SemaphoreType.DMA((N_DEV - 1,)),
            pltpu.SemaphoreType.DMA((N_DEV - 1,)),
            pltpu.SemaphoreType.DMA((N_DEV - 1,)),
        ],
        compiler_params=pltpu.CompilerParams(
            collective_id=0, vmem_limit_bytes=60 * 1024 * 1024),
    )(x, Wq, kc, vc, Wo)

    return out


# device time: 105563 ns/iter; 1.0570x vs baseline; 1.0570x over previous
import jax
import jax.numpy as jnp
from jax import lax
from jax.experimental import pallas as pl
from jax.experimental.pallas import tpu as pltpu

N_DEV = 8
SQ = 2048
D_MODEL = 1024
HQ_PER = 8
DH = 128
D_HEADS = HQ_PER * DH
BLK = SQ // N_DEV
WIN = 128
KW = 512
SCALE = 0.08838834764831843


def kernel(x, Wq, K_ext, V_ext, Wo):
    cdt = jnp.bfloat16

    kc = K_ext.reshape(SQ, HQ_PER * DH).astype(cdt)
    vc = V_ext.reshape(SQ, HQ_PER * DH).astype(cdt)

    def body(x_hbm, wq_hbm, k_hbm, v_hbm, wo_hbm, out_ref,
             xv, wqv, wov, wqb, wob, kb, vb,
             ctx_ref, send_ref, rs_ref, red_ref, ag_ref, acc_ref,
             load_sems, rs_send_sems, rs_recv_sems, ag_send_sems,
             ag_recv_sems):
        me_i = lax.axis_index("i")

        loads = [
            pltpu.make_async_copy(x_hbm.at[0], xv, load_sems.at[0]),
            pltpu.make_async_copy(
                wq_hbm.at[:, pl.ds(me_i * D_HEADS, D_HEADS)], wqv,
                load_sems.at[1]),
            pltpu.make_async_copy(
                wo_hbm.at[pl.ds(me_i * D_HEADS, D_HEADS), :], wov,
                load_sems.at[2]),
        ]
        for h in range(HQ_PER):
            loads.append(pltpu.make_async_copy(
                k_hbm.at[:, pl.ds(h * DH, DH)], kb.at[h], load_sems.at[3 + 2 * h]))
            loads.append(pltpu.make_async_copy(
                v_hbm.at[:, pl.ds(h * DH, DH)], vb.at[h], load_sems.at[4 + 2 * h]))
        for ld in loads:
            ld.start()

        bar = pltpu.get_barrier_semaphore()
        for j in range(1, N_DEV):
            pl.semaphore_signal(
                bar, inc=1,
                device_id=(lax.rem(me_i + j, N_DEV),),
                device_id_type=pl.DeviceIdType.MESH,
            )
        pl.semaphore_wait(bar, N_DEV - 1)

        loads[0].wait()
        loads[1].wait()
        wqb[...] = (wqv[...] * SCALE).astype(cdt)

        rs_rdmas = []
        for j in range(N_DEV):
            b = lax.rem(me_i + j, N_DEV)
            q0 = b * BLK
            kw = jnp.clip(q0 - WIN, 0, SQ - KW)
            kw = pl.multiple_of(kw, 128)
            xb = xv[pl.ds(q0, BLK), :].astype(cdt)
            qb = lax.dot_general(
                xb, wqb[...], (((1,), (0,)), ((), ())),
                preferred_element_type=jnp.float32)
            ri = lax.broadcasted_iota(jnp.int32, (BLK, KW), 0) + q0
            ci = lax.broadcasted_iota(jnp.int32, (BLK, KW), 1) + kw
            mask = jnp.abs(ri - ci) <= WIN
            for h in range(HQ_PER):
                if j == 0:
                    loads[3 + 2 * h].wait()
                    loads[4 + 2 * h].wait()
                qh = qb[:, h * DH:(h + 1) * DH].astype(cdt)
                ks = kb[h, pl.ds(kw, KW), :]
                s = lax.dot_general(
                    qh, ks, (((1,), (1,)), ((), ())),
                    preferred_element_type=jnp.float32)
                w = jnp.exp(jnp.where(mask, s, -1e9))
                recip = 1.0 / jnp.sum(w, axis=1, keepdims=True)
                vs = vb[h, pl.ds(kw, KW), :]
                ctxh = lax.dot_general(
                    w.astype(cdt), vs, (((1,), (0,)), ((), ())),
                    preferred_element_type=jnp.float32)
                ctx_ref[:, h * DH:(h + 1) * DH] = (ctxh * recip).astype(cdt)
            if j == 0:
                loads[2].wait()
                wob[...] = wov[...].astype(cdt)
            partial = lax.dot_general(
                ctx_ref[...], wob[...], (((1,), (0,)), ((), ())),
                preferred_element_type=jnp.float32)
            out_ref[0, pl.ds(q0, BLK), :] = partial
            if j == 0:
                continue
            send_ref[j - 1, :, :] = partial.astype(cdt)
            rdma = pltpu.make_async_remote_copy(
                src_ref=send_ref.at[j - 1],
                dst_ref=rs_ref.at[j - 1],
                send_sem=rs_send_sems.at[j - 1],
                recv_sem=rs_recv_sems.at[j - 1],
                device_id=(b,),
                device_id_type=pl.DeviceIdType.MESH,
            )
            rdma.start()
            rs_rdmas.append(rdma)

        acc_ref[...] = out_ref[0, pl.ds(me_i * BLK, BLK), :]
        for j in range(1, N_DEV):
            rs_rdmas[j - 1].wait_recv()
            acc_ref[...] += rs_ref[j - 1].astype(jnp.float32)
        out_ref[0, pl.ds(me_i * BLK, BLK), :] = acc_ref[...]
        red_ref[...] = acc_ref[...].astype(cdt)

        ag_rdmas = []
        for j in range(1, N_DEV):
            peer = lax.rem(me_i + j, N_DEV)
            rdma = pltpu.make_async_remote_copy(
                src_ref=red_ref,
                dst_ref=ag_ref.at[j - 1],
                send_sem=ag_send_sems.at[j - 1],
                recv_sem=ag_recv_sems.at[j - 1],
                device_id=(peer,),
                device_id_type=pl.DeviceIdType.MESH,
            )
            rdma.start()
            ag_rdmas.append(rdma)

        for j in range(1, N_DEV):
            ag_rdmas[j - 1].wait_recv()
            src_chunk = lax.rem(me_i + (N_DEV - j), N_DEV)
            out_ref[0, pl.ds(src_chunk * BLK, BLK), :] = ag_ref[j - 1].astype(jnp.float32)

        for r in rs_rdmas + ag_rdmas:
            r.wait_send()

    out = pl.pallas_call(
        body,
        out_shape=jax.ShapeDtypeStruct((1, SQ, D_MODEL), jnp.float32),
        in_specs=[pl.BlockSpec(memory_space=pltpu.MemorySpace.HBM)] * 5,
        out_specs=pl.BlockSpec(memory_space=pltpu.VMEM),
        scratch_shapes=[
            pltpu.VMEM((SQ, D_MODEL), jnp.float32),
            pltpu.VMEM((D_MODEL, D_HEADS), jnp.float32),
            pltpu.VMEM((D_HEADS, D_MODEL), jnp.float32),
            pltpu.VMEM((D_MODEL, D_HEADS), cdt),
            pltpu.VMEM((D_HEADS, D_MODEL), cdt),
            pltpu.VMEM((HQ_PER, SQ, DH), cdt),
            pltpu.VMEM((HQ_PER, SQ, DH), cdt),
            pltpu.VMEM((BLK, D_HEADS), cdt),
            pltpu.VMEM((N_DEV - 1, BLK, D_MODEL), cdt),
            pltpu.VMEM((N_DEV - 1, BLK, D_MODEL), cdt),
            pltpu.VMEM((BLK, D_MODEL), cdt),
            pltpu.VMEM((N_DEV - 1, BLK, D_MODEL), cdt),
            pltpu.VMEM((BLK, D_MODEL), jnp.float32),
            pltpu.SemaphoreType.DMA((3 + 2 * HQ_PER,)),
            pltpu.SemaphoreType.DMA((N_DEV - 1,)),
            pltpu.SemaphoreType.DMA((N_DEV - 1,)),
            pltpu.SemaphoreType.DMA((N_DEV - 1,)),
            pltpu.SemaphoreType.DMA((N_DEV - 1,)),
        ],
        compiler_params=pltpu.CompilerParams(
            collective_id=0, vmem_limit_bytes=60 * 1024 * 1024),
    )(x, Wq, kc, vc, Wo)

    return out


# device time: 101732 ns/iter; 1.0968x vs baseline; 1.0377x over previous
import jax
import jax.numpy as jnp
from jax import lax
from jax.experimental import pallas as pl
from jax.experimental.pallas import tpu as pltpu

N_DEV = 8
SQ = 2048
D_MODEL = 1024
HQ_PER = 8
DH = 128
D_HEADS = HQ_PER * DH
BLK = SQ // N_DEV
WIN = 128
KW = 512
SCALE = 0.08838834764831843


def kernel(x, Wq, K_ext, V_ext, Wo):
    cdt = jnp.bfloat16

    kc = K_ext.reshape(SQ, HQ_PER * DH).astype(cdt)
    vc = V_ext.reshape(SQ, HQ_PER * DH).astype(cdt)

    def body(x_hbm, wq_hbm, k_hbm, v_hbm, wo_hbm, out_ref,
             xv, wqv, wov, wqb, wob, kb, vb,
             ctx_ref, send_ref, rs_ref, red_ref, ag_ref, acc_ref,
             load_sems, rs_send_sems, rs_recv_sems, ag_send_sems,
             ag_recv_sems):
        me_i = lax.axis_index("i")

        loads = [
            pltpu.make_async_copy(x_hbm.at[0], xv, load_sems.at[0]),
            pltpu.make_async_copy(
                wq_hbm.at[:, pl.ds(me_i * D_HEADS, D_HEADS)], wqv,
                load_sems.at[1]),
            pltpu.make_async_copy(
                wo_hbm.at[pl.ds(me_i * D_HEADS, D_HEADS), :], wov,
                load_sems.at[2]),
        ]
        for h in range(HQ_PER):
            loads.append(pltpu.make_async_copy(
                k_hbm.at[:, pl.ds(h * DH, DH)], kb.at[h], load_sems.at[3 + 2 * h]))
            loads.append(pltpu.make_async_copy(
                v_hbm.at[:, pl.ds(h * DH, DH)], vb.at[h], load_sems.at[4 + 2 * h]))
        for ld in loads:
            ld.start()

        bar = pltpu.get_barrier_semaphore()
        for j in range(1, N_DEV):
            pl.semaphore_signal(
                bar, inc=1,
                device_id=(lax.rem(me_i + j, N_DEV),),
                device_id_type=pl.DeviceIdType.MESH,
            )
        pl.semaphore_wait(bar, N_DEV - 1)

        loads[0].wait()
        loads[1].wait()
        wqb[...] = (wqv[...] * SCALE).astype(cdt)

        rs_rdmas = []
        for idx, j in enumerate(list(range(1, N_DEV)) + [0]):
            first = idx == 0
            b = lax.rem(me_i + j, N_DEV)
            q0 = b * BLK
            kw = jnp.clip(q0 - WIN, 0, SQ - KW)
            kw = pl.multiple_of(kw, 128)
            xb = xv[pl.ds(q0, BLK), :].astype(cdt)
            qb = lax.dot_general(
                xb, wqb[...], (((1,), (0,)), ((), ())),
                preferred_element_type=jnp.float32)
            ri = lax.broadcasted_iota(jnp.int32, (BLK, KW), 0) + q0
            ci = lax.broadcasted_iota(jnp.int32, (BLK, KW), 1) + kw
            mask = jnp.abs(ri - ci) <= WIN
            for h in range(HQ_PER):
                if first:
                    loads[3 + 2 * h].wait()
                    loads[4 + 2 * h].wait()
                qh = qb[:, h * DH:(h + 1) * DH].astype(cdt)
                ks = kb[h, pl.ds(kw, KW), :]
                s = lax.dot_general(
                    qh, ks, (((1,), (1,)), ((), ())),
                    preferred_element_type=jnp.float32)
                w = jnp.exp(jnp.where(mask, s, -1e9))
                recip = 1.0 / jnp.sum(w, axis=1, keepdims=True)
                vs = vb[h, pl.ds(kw, KW), :]
                ctxh = lax.dot_general(
                    w.astype(cdt), vs, (((1,), (0,)), ((), ())),
                    preferred_element_type=jnp.float32)
                ctx_ref[:, h * DH:(h + 1) * DH] = (ctxh * recip).astype(cdt)
            if first:
                loads[2].wait()
                wob[...] = wov[...].astype(cdt)
            partial = lax.dot_general(
                ctx_ref[...], wob[...], (((1,), (0,)), ((), ())),
                preferred_element_type=jnp.float32)
            out_ref[0, pl.ds(q0, BLK), :] = partial
            if j == 0:
                continue
            send_ref[j - 1, :, :] = partial.astype(cdt)
            rdma = pltpu.make_async_remote_copy(
                src_ref=send_ref.at[j - 1],
                dst_ref=rs_ref.at[j - 1],
                send_sem=rs_send_sems.at[j - 1],
                recv_sem=rs_recv_sems.at[j - 1],
                device_id=(b,),
                device_id_type=pl.DeviceIdType.MESH,
            )
            rdma.start()
            rs_rdmas.append(rdma)

        acc_ref[...] = out_ref[0, pl.ds(me_i * BLK, BLK), :]
        for j in range(1, N_DEV):
            rs_rdmas[j - 1].wait_recv()
            acc_ref[...] += rs_ref[j - 1].astype(jnp.float32)
        out_ref[0, pl.ds(me_i * BLK, BLK), :] = acc_ref[...]
        red_ref[...] = acc_ref[...].astype(cdt)

        ag_rdmas = []
        for j in range(1, N_DEV):
            peer = lax.rem(me_i + j, N_DEV)
            rdma = pltpu.make_async_remote_copy(
                src_ref=red_ref,
                dst_ref=ag_ref.at[j - 1],
                send_sem=ag_send_sems.at[j - 1],
                recv_sem=ag_recv_sems.at[j - 1],
                device_id=(peer,),
                device_id_type=pl.DeviceIdType.MESH,
            )
            rdma.start()
            ag_rdmas.append(rdma)

        for j in range(1, N_DEV):
            ag_rdmas[j - 1].wait_recv()
            src_chunk = lax.rem(me_i + (N_DEV - j), N_DEV)
            out_ref[0, pl.ds(src_chunk * BLK, BLK), :] = ag_ref[j - 1].astype(jnp.float32)

        for r in rs_rdmas + ag_rdmas:
            r.wait_send()

    out = pl.pallas_call(
        body,
        out_shape=jax.ShapeDtypeStruct((1, SQ, D_MODEL), jnp.float32),
        in_specs=[pl.BlockSpec(memory_space=pltpu.MemorySpace.HBM)] * 5,
        out_specs=pl.BlockSpec(memory_space=pltpu.VMEM),
        scratch_shapes=[
            pltpu.VMEM((SQ, D_MODEL), jnp.float32),
            pltpu.VMEM((D_MODEL, D_HEADS), jnp.float32),
            pltpu.VMEM((D_HEADS, D_MODEL), jnp.float32),
            pltpu.VMEM((D_MODEL, D_HEADS), cdt),
            pltpu.VMEM((D_HEADS, D_MODEL), cdt),
            pltpu.VMEM((HQ_PER, SQ, DH), cdt),
            pltpu.VMEM((HQ_PER, SQ, DH), cdt),
            pltpu.VMEM((BLK, D_HEADS), cdt),
            pltpu.VMEM((N_DEV - 1, BLK, D_MODEL), cdt),
            pltpu.VMEM((N_DEV - 1, BLK, D_MODEL), cdt),
            pltpu.VMEM((BLK, D_MODEL), cdt),
            pltpu.VMEM((N_DEV - 1, BLK, D_MODEL), cdt),
            pltpu.VMEM((BLK, D_MODEL), jnp.float32),
            pltpu.SemaphoreType.DMA((3 + 2 * HQ_PER,)),
            pltpu.SemaphoreType.DMA((N_DEV - 1,)),
            pltpu.SemaphoreType.DMA((N_DEV - 1,)),
            pltpu.SemaphoreType.DMA((N_DEV - 1,)),
            pltpu.SemaphoreType.DMA((N_DEV - 1,)),
        ],
        compiler_params=pltpu.CompilerParams(
            collective_id=0, vmem_limit_bytes=60 * 1024 * 1024),
    )(x, Wq, kc, vc, Wo)

    return out
